# Initial kernel scaffold; baseline (speedup 1.0000x reference)
import jax
import jax.numpy as jnp
from jax.experimental import pallas as pl


def kernel(x, W, b):
    raise NotImplementedError("write your pallas kernel here")


# speedup vs baseline: 4.6959x; 4.6959x over previous
import jax
import jax.numpy as jnp
from jax import lax
from jax.experimental import pallas as pl

_D = 768
_NE = 64
_BT = 512


def _gate_rows(s):
    ncols = s.shape[-1]
    col = lax.broadcasted_iota(jnp.int32, s.shape, 1)
    m1 = jnp.max(s, axis=-1, keepdims=True)
    i1 = jnp.min(jnp.where(s == m1, col, ncols), axis=-1, keepdims=True)
    s2 = jnp.where(col == i1, -jnp.inf, s)
    m2 = jnp.max(s2, axis=-1, keepdims=True)
    i2 = jnp.min(jnp.where(s2 == m2, col, ncols), axis=-1, keepdims=True)
    mask = (col == i1) | (col == i2)
    e = jnp.exp(s - m1)
    z = jnp.sum(e, axis=-1, keepdims=True)
    denom = 1.0 + jnp.exp(m2 - m1) + 1e-8 * z
    return jnp.where(mask, e, 0.0) / denom


def _fused_body(x_ref, w_ref, b_ref, o_ref):
    s = lax.dot_general(
        x_ref[...], w_ref[...],
        (((1,), (1,)), ((), ())),
        preferred_element_type=jnp.float32,
    )
    s = s + b_ref[...]
    o_ref[...] = _gate_rows(s)


def kernel(x, W, b):
    t = x.shape[0]
    b2 = b.reshape(1, _NE)
    return pl.pallas_call(
        _fused_body,
        grid=(t // _BT,),
        in_specs=[
            pl.BlockSpec((_BT, _D), lambda i: (i, 0)),
            pl.BlockSpec((_NE, _D), lambda i: (0, 0)),
            pl.BlockSpec((1, _NE), lambda i: (0, 0)),
        ],
        out_specs=pl.BlockSpec((_BT, _NE), lambda i: (i, 0)),
        out_shape=jax.ShapeDtypeStruct((t, _NE), jnp.float32),
    )(x, W, b2)


# speedup vs baseline: 6.1869x; 1.3175x over previous
import jax
import jax.numpy as jnp
from jax import lax
from jax.experimental import pallas as pl

_D = 768
_NE = 64
_BT = 1024


def _gate_rows(s):
    ncols = s.shape[-1]
    col = lax.broadcasted_iota(jnp.int32, s.shape, 1)
    m1 = jnp.max(s, axis=-1, keepdims=True)
    i1 = jnp.min(jnp.where(s == m1, col, ncols), axis=-1, keepdims=True)
    s2 = jnp.where(col == i1, -jnp.inf, s)
    m2 = jnp.max(s2, axis=-1, keepdims=True)
    i2 = jnp.min(jnp.where(s2 == m2, col, ncols), axis=-1, keepdims=True)
    mask = (col == i1) | (col == i2)
    e = jnp.exp(s - m1)
    z = jnp.sum(e, axis=-1, keepdims=True)
    denom = 1.0 + jnp.exp(m2 - m1) + 1e-8 * z
    return jnp.where(mask, e, 0.0) / denom


def _fused_body(x_ref, w_ref, b_ref, o_ref):
    s = lax.dot_general(
        x_ref[...], w_ref[...],
        (((1,), (1,)), ((), ())),
        preferred_element_type=jnp.float32,
    )
    s = s + b_ref[...]
    o_ref[...] = _gate_rows(s)


def kernel(x, W, b):
    t = x.shape[0]
    b2 = b.reshape(1, _NE)
    return pl.pallas_call(
        _fused_body,
        grid=(t // _BT,),
        in_specs=[
            pl.BlockSpec((_BT, _D), lambda i: (i, 0)),
            pl.BlockSpec((_NE, _D), lambda i: (0, 0)),
            pl.BlockSpec((1, _NE), lambda i: (0, 0)),
        ],
        out_specs=pl.BlockSpec((_BT, _NE), lambda i: (i, 0)),
        out_shape=jax.ShapeDtypeStruct((t, _NE), jnp.float32),
    )(x, W, b2)


# speedup vs baseline: 7.0326x; 1.1367x over previous
import jax
import jax.numpy as jnp
from jax import lax
from jax.experimental import pallas as pl

_D = 768
_NE = 64
_BT = 2048


def _gate_rows(s):
    ncols = s.shape[-1]
    col = lax.broadcasted_iota(jnp.int32, s.shape, 1)
    m1 = jnp.max(s, axis=-1, keepdims=True)
    i1 = jnp.min(jnp.where(s == m1, col, ncols), axis=-1, keepdims=True)
    s2 = jnp.where(col == i1, -jnp.inf, s)
    m2 = jnp.max(s2, axis=-1, keepdims=True)
    i2 = jnp.min(jnp.where(s2 == m2, col, ncols), axis=-1, keepdims=True)
    mask = (col == i1) | (col == i2)
    e = jnp.exp(s - m1)
    z = jnp.sum(e, axis=-1, keepdims=True)
    denom = 1.0 + jnp.exp(m2 - m1) + 1e-8 * z
    return jnp.where(mask, e, 0.0) / denom


def _fused_body(x_ref, w_ref, b_ref, o_ref):
    s = lax.dot_general(
        x_ref[...], w_ref[...],
        (((1,), (1,)), ((), ())),
        preferred_element_type=jnp.float32,
    )
    s = s + b_ref[...]
    o_ref[...] = _gate_rows(s)


def kernel(x, W, b):
    t = x.shape[0]
    b2 = b.reshape(1, _NE)
    return pl.pallas_call(
        _fused_body,
        grid=(t // _BT,),
        in_specs=[
            pl.BlockSpec((_BT, _D), lambda i: (i, 0)),
            pl.BlockSpec((_NE, _D), lambda i: (0, 0)),
            pl.BlockSpec((1, _NE), lambda i: (0, 0)),
        ],
        out_specs=pl.BlockSpec((_BT, _NE), lambda i: (i, 0)),
        out_shape=jax.ShapeDtypeStruct((t, _NE), jnp.float32),
    )(x, W, b2)


# speedup vs baseline: 7.6349x; 1.0856x over previous
import jax
import jax.numpy as jnp
from jax import lax
from jax.experimental import pallas as pl

_D = 768
_NE = 64
_BT = 4096


def _gate_rows(s):
    ncols = s.shape[-1]
    col = lax.broadcasted_iota(jnp.int32, s.shape, 1)
    m1 = jnp.max(s, axis=-1, keepdims=True)
    i1 = jnp.min(jnp.where(s == m1, col, ncols), axis=-1, keepdims=True)
    s2 = jnp.where(col == i1, -jnp.inf, s)
    m2 = jnp.max(s2, axis=-1, keepdims=True)
    i2 = jnp.min(jnp.where(s2 == m2, col, ncols), axis=-1, keepdims=True)
    mask = (col == i1) | (col == i2)
    e = jnp.exp(s - m1)
    z = jnp.sum(e, axis=-1, keepdims=True)
    denom = 1.0 + jnp.exp(m2 - m1) + 1e-8 * z
    return jnp.where(mask, e, 0.0) / denom


def _fused_body(x_ref, w_ref, b_ref, o_ref):
    s = lax.dot_general(
        x_ref[...], w_ref[...],
        (((1,), (1,)), ((), ())),
        preferred_element_type=jnp.float32,
    )
    s = s + b_ref[...]
    o_ref[...] = _gate_rows(s)


def kernel(x, W, b):
    t = x.shape[0]
    b2 = b.reshape(1, _NE)
    return pl.pallas_call(
        _fused_body,
        grid=(t // _BT,),
        in_specs=[
            pl.BlockSpec((_BT, _D), lambda i: (i, 0)),
            pl.BlockSpec((_NE, _D), lambda i: (0, 0)),
            pl.BlockSpec((1, _NE), lambda i: (0, 0)),
        ],
        out_specs=pl.BlockSpec((_BT, _NE), lambda i: (i, 0)),
        out_shape=jax.ShapeDtypeStruct((t, _NE), jnp.float32),
    )(x, W, b2)


# speedup vs baseline: 8.5552x; 1.1205x over previous
import jax
import jax.numpy as jnp
from jax import lax
from jax.experimental import pallas as pl

_D = 768
_NE = 64
_BT = 4096


def _gate_rows(s):
    col = lax.broadcasted_iota(jnp.int32, s.shape, 1)
    colf = col.astype(jnp.float32)
    ui = lax.bitcast_convert_type(s, jnp.int32)
    idxbits = jnp.where(s < 0.0, col, _NE - 1 - col)
    kf = lax.bitcast_convert_type((ui & -_NE) | idxbits, jnp.float32)
    k1 = jnp.max(kf, axis=-1, keepdims=True)
    is1 = kf == k1
    s2 = jnp.where(is1, -jnp.inf, s)
    m2 = jnp.max(s2, axis=-1, keepdims=True)
    i2 = jnp.min(jnp.where(s2 == m2, colf, float(_NE)), axis=-1, keepdims=True)
    mask = is1 | (colf == i2)
    e = jnp.exp(s - k1)
    c = jnp.where(mask, 1.0 + 1e-8, 1e-8)
    denom = jnp.sum(e * c, axis=-1, keepdims=True)
    return jnp.where(mask, e, 0.0) * (1.0 / denom)


def _fused_body(x_ref, w_ref, b_ref, o_ref):
    s = lax.dot_general(
        x_ref[...], w_ref[...],
        (((1,), (1,)), ((), ())),
        preferred_element_type=jnp.float32,
    )
    s = s + b_ref[...]
    o_ref[...] = _gate_rows(s)


def kernel(x, W, b):
    t = x.shape[0]
    b2 = b.reshape(1, _NE)
    return pl.pallas_call(
        _fused_body,
        grid=(t // _BT,),
        in_specs=[
            pl.BlockSpec((_BT, _D), lambda i: (i, 0)),
            pl.BlockSpec((_NE, _D), lambda i: (0, 0)),
            pl.BlockSpec((1, _NE), lambda i: (0, 0)),
        ],
        out_specs=pl.BlockSpec((_BT, _NE), lambda i: (i, 0)),
        out_shape=jax.ShapeDtypeStruct((t, _NE), jnp.float32),
    )(x, W, b2)
